# Initial kernel scaffold; baseline (speedup 1.0000x reference)
import jax
import jax.numpy as jnp
from jax.experimental import pallas as pl


def kernel(tokens, table):
    raise NotImplementedError("write your pallas kernel here")


# speedup vs baseline: 2.8131x; 2.8131x over previous
import functools

import jax
import jax.numpy as jnp
from jax import lax
from jax.experimental import pallas as pl
from jax.experimental.pallas import tpu as pltpu
from jax.experimental.pallas import tpu_sc as plsc

BATCH = 16384
SEQ = 200
HIDDEN = 128
NTOK = BATCH * SEQ

NC = 2
NS = 16
NW = NC * NS
PER_W = NTOK // NW

IDXW = 128
CHUNK = 512
GPC = CHUNK // IDXW
NCHUNK = PER_W // CHUNK

ONES_LEN = 2048

_mesh = plsc.VectorSubcoreMesh(core_axis_name="c", subcore_axis_name="s")


@functools.partial(
    pl.kernel,
    out_type=(
        jax.ShapeDtypeStruct((NTOK, HIDDEN), jnp.float32),
        jax.ShapeDtypeStruct((NTOK,), jnp.float32),
    ),
    mesh=_mesh,
    scratch_types=[
        pltpu.VMEM((GPC, IDXW), jnp.int32),
        pltpu.VMEM((CHUNK, HIDDEN), jnp.float32),
        pltpu.VMEM((ONES_LEN,), jnp.float32),
        pltpu.SemaphoreType.DMA,
    ],
)
def _embed_sc(tok_hbm, table_hbm, out_hbm, mask_hbm, idx_v, rows_v, ones_v, sem):
    wid = lax.axis_index("s") * NC + lax.axis_index("c")
    base = wid * PER_W

    base_rows = wid * (PER_W // IDXW)

    def chunk_body(g, carry):
        off = base + g * CHUNK
        pltpu.sync_copy(tok_hbm.at[pl.ds(base_rows + g * GPC, GPC)], idx_v)
        for j in range(GPC):
            pltpu.async_copy(
                table_hbm.at[idx_v.at[j]],
                rows_v.at[pl.ds(j * IDXW, IDXW)],
                sem,
            ).wait()
        pltpu.sync_copy(rows_v, out_hbm.at[pl.ds(off, CHUNK)])
        return carry

    lax.fori_loop(0, NCHUNK, chunk_body, 0)

    def fill_body(i, carry):
        ones_v[pl.ds(i * 16, 16)] = jnp.ones((16,), jnp.float32)
        return carry

    lax.fori_loop(0, ONES_LEN // 16, fill_body, 0)

    def mask_body(g, carry):
        pltpu.sync_copy(ones_v, mask_hbm.at[pl.ds(base + g * ONES_LEN, ONES_LEN)])
        return carry

    lax.fori_loop(0, PER_W // ONES_LEN, mask_body, 0)


def kernel(tokens, table):
    out_flat, mask_flat = _embed_sc(tokens.reshape(NTOK // IDXW, IDXW), table)
    return (
        out_flat.reshape(BATCH, SEQ, HIDDEN),
        mask_flat.reshape(BATCH, SEQ),
    )


# speedup vs baseline: 2.8750x; 1.0220x over previous
import functools

import jax
import jax.numpy as jnp
from jax import lax
from jax.experimental import pallas as pl
from jax.experimental.pallas import tpu as pltpu
from jax.experimental.pallas import tpu_sc as plsc

BATCH = 16384
SEQ = 200
HIDDEN = 128
NTOK = BATCH * SEQ

NC = 2
NS = 16
NW = NC * NS
PER_W = NTOK // NW

IDXW = 128
CHUNK = 256
GPC = CHUNK // IDXW
NCHUNK = PER_W // CHUNK
NPAIR = NCHUNK // 2

ONES_LEN = 2048

_mesh = plsc.VectorSubcoreMesh(core_axis_name="c", subcore_axis_name="s")


@functools.partial(
    pl.kernel,
    out_type=(
        jax.ShapeDtypeStruct((NTOK, HIDDEN), jnp.float32),
        jax.ShapeDtypeStruct((NTOK,), jnp.float32),
    ),
    mesh=_mesh,
    scratch_types=[
        pltpu.VMEM((GPC, IDXW), jnp.int32),
        pltpu.VMEM((GPC, IDXW), jnp.int32),
        pltpu.VMEM((CHUNK, HIDDEN), jnp.float32),
        pltpu.VMEM((CHUNK, HIDDEN), jnp.float32),
        pltpu.VMEM((ONES_LEN,), jnp.float32),
        pltpu.SemaphoreType.DMA,
        pltpu.SemaphoreType.DMA,
        pltpu.SemaphoreType.DMA,
        pltpu.SemaphoreType.DMA,
    ],
)
def _embed_sc(tok_hbm, table_hbm, out_hbm, mask_hbm,
              idx0, idx1, rows0, rows1, ones_v,
              gsem0, gsem1, ssem0, ssem1):
    wid = lax.axis_index("s") * NC + lax.axis_index("c")
    base = wid * PER_W
    base_rows = wid * (PER_W // IDXW)

    def gather_chunk(g, idx_s, rows_s, gs_s):
        pltpu.sync_copy(tok_hbm.at[pl.ds(base_rows + g * GPC, GPC)], idx_s)
        descs = [
            pltpu.async_copy(
                table_hbm.at[idx_s.at[j]],
                rows_s.at[pl.ds(j * IDXW, IDXW)],
                gs_s,
            )
            for j in range(GPC)
        ]
        for d in descs:
            d.wait()

    def fire_store(g, rows_s, ss_s):
        pltpu.async_copy(rows_s, out_hbm.at[pl.ds(base + g * CHUNK, CHUNK)], ss_s)

    def wait_store(g, rows_s, ss_s):
        pltpu.make_async_copy(
            rows_s, out_hbm.at[pl.ds(base + g * CHUNK, CHUNK)], ss_s
        ).wait()

    gather_chunk(0, idx0, rows0, gsem0)
    fire_store(0, rows0, ssem0)
    gather_chunk(1, idx1, rows1, gsem1)
    fire_store(1, rows1, ssem1)

    def pair_body(t, carry):
        g = 2 * t
        wait_store(g - 2, rows0, ssem0)
        gather_chunk(g, idx0, rows0, gsem0)
        fire_store(g, rows0, ssem0)
        wait_store(g - 1, rows1, ssem1)
        gather_chunk(g + 1, idx1, rows1, gsem1)
        fire_store(g + 1, rows1, ssem1)
        return carry

    lax.fori_loop(1, NPAIR, pair_body, 0)

    wait_store(NCHUNK - 2, rows0, ssem0)
    wait_store(NCHUNK - 1, rows1, ssem1)

    def fill_body(i, carry):
        ones_v[pl.ds(i * 16, 16)] = jnp.ones((16,), jnp.float32)
        return carry

    lax.fori_loop(0, ONES_LEN // 16, fill_body, 0)

    def mask_body(g, carry):
        pltpu.sync_copy(ones_v, mask_hbm.at[pl.ds(base + g * ONES_LEN, ONES_LEN)])
        return carry

    lax.fori_loop(0, PER_W // ONES_LEN, mask_body, 0)


def kernel(tokens, table):
    out_flat, mask_flat = _embed_sc(tokens.reshape(NTOK // IDXW, IDXW), table)
    return (
        out_flat.reshape(BATCH, SEQ, HIDDEN),
        mask_flat.reshape(BATCH, SEQ),
    )


# speedup vs baseline: 14.4627x; 5.0305x over previous
import functools

import jax
import jax.numpy as jnp
from jax import lax
from jax.experimental import pallas as pl
from jax.experimental.pallas import tpu as pltpu
from jax.experimental.pallas import tpu_sc as plsc

BATCH = 16384
SEQ = 200
HIDDEN = 128
VOCAB = 100
NTOK = BATCH * SEQ

NC = 2
NS = 16
NW = NC * NS
PER_W = NTOK // NW

IDXW = 128
CHUNK = 256
GPC = CHUNK // IDXW
NCHUNK = PER_W // CHUNK
NPAIR = NCHUNK // 2

ONES_LEN = 2048

_mesh = plsc.VectorSubcoreMesh(core_axis_name="c", subcore_axis_name="s")


@functools.partial(
    pl.kernel,
    out_type=(
        jax.ShapeDtypeStruct((NTOK, HIDDEN), jnp.float32),
        jax.ShapeDtypeStruct((NTOK,), jnp.float32),
    ),
    mesh=_mesh,
    scratch_types=[
        pltpu.VMEM((GPC, IDXW), jnp.int32),
        pltpu.VMEM((GPC, IDXW), jnp.int32),
        pltpu.VMEM((CHUNK, HIDDEN), jnp.float32),
        pltpu.VMEM((CHUNK, HIDDEN), jnp.float32),
        pltpu.VMEM((ONES_LEN,), jnp.float32),
        pltpu.VMEM_SHARED((VOCAB, HIDDEN), jnp.float32),
        pltpu.SemaphoreType.DMA,
        pltpu.SemaphoreType.DMA,
        pltpu.SemaphoreType.DMA,
        pltpu.SemaphoreType.DMA,
    ],
)
def _embed_sc(tok_hbm, table_hbm, out_hbm, mask_hbm,
              idx0, idx1, rows0, rows1, ones_v, table_sp,
              gsem0, gsem1, ssem0, ssem1):
    sid = lax.axis_index("s")
    wid = sid * NC + lax.axis_index("c")
    base = wid * PER_W
    base_rows = wid * (PER_W // IDXW)

    @pl.when(sid == 0)
    def _():
        pltpu.sync_copy(table_hbm, table_sp)

    plsc.subcore_barrier()

    def gather_chunk(g, idx_s, rows_s, gs_s):
        pltpu.sync_copy(tok_hbm.at[pl.ds(base_rows + g * GPC, GPC)], idx_s)
        descs = [
            pltpu.async_copy(
                table_sp.at[idx_s.at[j]],
                rows_s.at[pl.ds(j * IDXW, IDXW)],
                gs_s,
            )
            for j in range(GPC)
        ]
        for d in descs:
            d.wait()

    def fire_store(g, rows_s, ss_s):
        pltpu.async_copy(rows_s, out_hbm.at[pl.ds(base + g * CHUNK, CHUNK)], ss_s)

    def wait_store(g, rows_s, ss_s):
        pltpu.make_async_copy(
            rows_s, out_hbm.at[pl.ds(base + g * CHUNK, CHUNK)], ss_s
        ).wait()

    gather_chunk(0, idx0, rows0, gsem0)
    fire_store(0, rows0, ssem0)
    gather_chunk(1, idx1, rows1, gsem1)
    fire_store(1, rows1, ssem1)

    def pair_body(t, carry):
        g = 2 * t
        wait_store(g - 2, rows0, ssem0)
        gather_chunk(g, idx0, rows0, gsem0)
        fire_store(g, rows0, ssem0)
        wait_store(g - 1, rows1, ssem1)
        gather_chunk(g + 1, idx1, rows1, gsem1)
        fire_store(g + 1, rows1, ssem1)
        return carry

    lax.fori_loop(1, NPAIR, pair_body, 0)

    wait_store(NCHUNK - 2, rows0, ssem0)
    wait_store(NCHUNK - 1, rows1, ssem1)

    def fill_body(i, carry):
        ones_v[pl.ds(i * 16, 16)] = jnp.ones((16,), jnp.float32)
        return carry

    lax.fori_loop(0, ONES_LEN // 16, fill_body, 0)

    def mask_body(g, carry):
        pltpu.sync_copy(ones_v, mask_hbm.at[pl.ds(base + g * ONES_LEN, ONES_LEN)])
        return carry

    lax.fori_loop(0, PER_W // ONES_LEN, mask_body, 0)


def kernel(tokens, table):
    out_flat, mask_flat = _embed_sc(tokens.reshape(NTOK // IDXW, IDXW), table)
    return (
        out_flat.reshape(BATCH, SEQ, HIDDEN),
        mask_flat.reshape(BATCH, SEQ),
    )
